# baseline (speedup 1.0000x reference)
import functools

import jax
import jax.numpy as jnp
from jax import lax
from jax.experimental import pallas as pl
from jax.experimental.pallas import tpu as pltpu
from jax.experimental.pallas import tpu_sc as plsc

NUM_CLASSES = 1000000
HIDDEN_SIZE = 64
BATCH = 16384
TABLE_ROWS = NUM_CLASSES + 1

_NC = 2
_NS = 16
_NW = _NC * _NS
_B_PER_W = BATCH // _NW
_CHUNK = 128
_NCHUNK = _B_PER_W // _CHUNK


def _make_gather():
    mesh = plsc.VectorSubcoreMesh(core_axis_name="c", subcore_axis_name="s")

    @functools.partial(
        pl.kernel,
        mesh=mesh,
        out_type=jax.ShapeDtypeStruct((BATCH, HIDDEN_SIZE), jnp.float32),
        compiler_params=pltpu.CompilerParams(use_tc_tiling_on_sc=False),
        scratch_types=[
            pltpu.VMEM((_NCHUNK, _CHUNK), jnp.int32),
            pltpu.VMEM((_B_PER_W, HIDDEN_SIZE), jnp.float32),
            pltpu.SemaphoreType.DMA,
        ],
    )
    def gather_kernel(table_hbm, idx_hbm, out_hbm, idx_v, rows_v, sem):
        wid = lax.axis_index("s") * _NC + lax.axis_index("c")
        base = wid * _B_PER_W
        pltpu.sync_copy(idx_hbm.at[wid], idx_v)
        copies = []
        for j in range(_NCHUNK):
            copies.append(
                pltpu.async_copy(
                    table_hbm.at[idx_v.at[j]],
                    rows_v.at[pl.ds(j * _CHUNK, _CHUNK)],
                    sem,
                )
            )
        for c in copies:
            c.wait()
        pltpu.sync_copy(rows_v, out_hbm.at[pl.ds(base, _B_PER_W)])

    return gather_kernel


_gather = _make_gather()


def kernel(labels, embedding_table):
    idx = labels.astype(jnp.int32).reshape(_NW, _NCHUNK, _CHUNK)
    return _gather(embedding_table, idx)


# speedup vs baseline: 1.0272x; 1.0272x over previous
import functools

import jax
import jax.numpy as jnp
from jax import lax
from jax.experimental import pallas as pl
from jax.experimental.pallas import tpu as pltpu
from jax.experimental.pallas import tpu_sc as plsc

NUM_CLASSES = 1000000
HIDDEN_SIZE = 64
BATCH = 16384
TABLE_ROWS = NUM_CLASSES + 1

_NC = 2
_NS = 16
_NW = _NC * _NS
_B_PER_W = BATCH // _NW
_K = 16
_NROUND = _B_PER_W // _K


def _make_gather():
    mesh = plsc.VectorSubcoreMesh(core_axis_name="c", subcore_axis_name="s")

    @functools.partial(
        pl.kernel,
        mesh=mesh,
        out_type=jax.ShapeDtypeStruct((BATCH, HIDDEN_SIZE), jnp.float32),
        compiler_params=pltpu.CompilerParams(use_tc_tiling_on_sc=True),
        scratch_types=[
            pltpu.VMEM((_B_PER_W,), jnp.int32),
            pltpu.SemaphoreType.DMA,
        ],
    )
    def gather_kernel(table_hbm, lab_hbm, out_hbm, lab_v, sem):
        wid = lax.axis_index("s") * _NC + lax.axis_index("c")
        base = wid * _B_PER_W
        pltpu.sync_copy(lab_hbm.at[pl.ds(base, _B_PER_W)], lab_v)

        @pl.loop(0, _NROUND)
        def _round(r):
            lv = lab_v[pl.ds(r * _K, _K)]
            copies = []
            for t in range(_K):
                copies.append(
                    pltpu.async_copy(
                        table_hbm.at[pl.ds(lv[t], 1)],
                        out_hbm.at[pl.ds(base + r * _K + t, 1)],
                        sem,
                    )
                )
            for c in copies:
                c.wait()

    return gather_kernel


_gather = _make_gather()


def kernel(labels, embedding_table):
    return _gather(embedding_table, labels.astype(jnp.int32))


# speedup vs baseline: 1.0290x; 1.0018x over previous
import functools

import jax
import jax.numpy as jnp
from jax import lax
from jax.experimental import pallas as pl
from jax.experimental.pallas import tpu as pltpu
from jax.experimental.pallas import tpu_sc as plsc

NUM_CLASSES = 1000000
HIDDEN_SIZE = 64
BATCH = 16384
TABLE_ROWS = NUM_CLASSES + 1

_NC = 2
_NS = 16
_NW = _NC * _NS
_B_PER_W = BATCH // _NW
_UNROLL = 16


def _make_gather():
    mesh = plsc.VectorSubcoreMesh(core_axis_name="c", subcore_axis_name="s")

    @functools.partial(
        pl.kernel,
        mesh=mesh,
        out_type=jax.ShapeDtypeStruct((BATCH, HIDDEN_SIZE), jnp.float32),
        compiler_params=pltpu.CompilerParams(use_tc_tiling_on_sc=True),
        scratch_types=[
            pltpu.VMEM((_B_PER_W,), jnp.int32),
            pltpu.SemaphoreType.DMA,
        ],
    )
    def gather_kernel(table_hbm, lab_hbm, out_hbm, lab_v, sem):
        wid = lax.axis_index("s") * _NC + lax.axis_index("c")
        base = wid * _B_PER_W
        pltpu.sync_copy(lab_hbm.at[pl.ds(base, _B_PER_W)], lab_v)

        @pl.loop(0, _B_PER_W // _UNROLL)
        def _blk(b):
            i0 = b * _UNROLL
            lv = lab_v[pl.ds(i0, _UNROLL)]
            for t in range(_UNROLL):
                pltpu.async_copy(
                    table_hbm.at[pl.ds(lv[t], 1)],
                    out_hbm.at[pl.ds(base + i0 + t, 1)],
                    sem,
                )

        pltpu.make_async_copy(
            out_hbm.at[pl.ds(base, _B_PER_W)],
            out_hbm.at[pl.ds(base, _B_PER_W)],
            sem,
        ).wait()

    return gather_kernel


_gather = _make_gather()


def kernel(labels, embedding_table):
    return _gather(embedding_table, labels.astype(jnp.int32))
